# baseline (speedup 1.0000x reference)
import functools

import jax
import jax.numpy as jnp
from jax.experimental import pallas as pl
from jax.experimental.pallas import tpu as pltpu

N = 10000
E = 160000
F_IN = 256
H = 512
C = 7
NUM_GRAPHS = 8

BM_BASIS = 128
G_MAX = (N + C * (BM_BASIS - 1) + BM_BASIS - 1) // BM_BASIS
BM_MM = 1000


def _basis_mm_kernel(bc_ref, x_ref, b_ref, o_ref):
    del bc_ref
    o_ref[...] = jnp.dot(x_ref[...], b_ref[0], preferred_element_type=jnp.float32)


def _grouped_basis_matmul(x_g, basis_weights, block_comm):
    grid_spec = pltpu.PrefetchScalarGridSpec(
        num_scalar_prefetch=1,
        grid=(G_MAX,),
        in_specs=[
            pl.BlockSpec((BM_BASIS, F_IN), lambda i, bc: (i, 0)),
            pl.BlockSpec((1, F_IN, H), lambda i, bc: (bc[i], 0, 0)),
        ],
        out_specs=pl.BlockSpec((BM_BASIS, H), lambda i, bc: (i, 0)),
    )
    return pl.pallas_call(
        _basis_mm_kernel,
        grid_spec=grid_spec,
        out_shape=jax.ShapeDtypeStruct((G_MAX * BM_BASIS, H), jnp.float32),
    )(block_comm, x_g, basis_weights)


def _mm_kernel(a_ref, b_ref, o_ref):
    o_ref[...] = jnp.dot(a_ref[...], b_ref[...], preferred_element_type=jnp.float32)


def _matmul(a, b):
    m, k = a.shape
    k2, n = b.shape
    return pl.pallas_call(
        _mm_kernel,
        grid=(m // BM_MM,),
        in_specs=[pl.BlockSpec((BM_MM, k), lambda i: (i, 0)),
                  pl.BlockSpec((k, n), lambda i: (0, 0))],
        out_specs=pl.BlockSpec((BM_MM, n), lambda i: (i, 0)),
        out_shape=jax.ShapeDtypeStruct((m, n), jnp.float32),
    )(a, b)


def _basis_transform(x, community_assignments, basis_weights):
    ca = community_assignments.astype(jnp.int32)
    counts = jnp.bincount(ca, length=C)
    nb = (counts + BM_BASIS - 1) // BM_BASIS
    cum_nb = jnp.cumsum(nb)
    block_comm = jnp.searchsorted(cum_nb, jnp.arange(G_MAX), side='right')
    block_comm = jnp.minimum(block_comm, C - 1).astype(jnp.int32)
    perm = jnp.argsort(ca, stable=True)
    cstart = jnp.concatenate([jnp.zeros((1,), jnp.int32),
                              jnp.cumsum(counts)[:-1].astype(jnp.int32)])
    pstart = jnp.concatenate([jnp.zeros((1,), jnp.int32),
                              (jnp.cumsum(nb)[:-1] * BM_BASIS).astype(jnp.int32)])
    slots = jnp.arange(G_MAX * BM_BASIS)
    sc = jnp.minimum(jnp.searchsorted(jnp.cumsum(nb) * BM_BASIS, slots, side='right'), C - 1)
    local = slots - pstart[sc]
    src_pos = cstart[sc] + jnp.minimum(local, jnp.maximum(counts[sc] - 1, 0))
    gather_idx = perm[src_pos]
    x_g = x[gather_idx]
    h0_pad = _grouped_basis_matmul(x_g, basis_weights, block_comm)
    node_slot = jnp.zeros((N,), jnp.int32).at[perm].set(
        (pstart[ca[perm]] + (jnp.arange(N) - cstart[ca[perm]].astype(jnp.int32))).astype(jnp.int32))
    return h0_pad[node_slot]


def kernel(x, edge_index, edge_weight, community_assignments, batch,
           basis_weights, W1, b1, W2, b2, pool_W, pool_b,
           fc1_W, fc1_b, fc2_W, fc2_b):
    num_nodes = x.shape[0]
    src, dst = edge_index[0], edge_index[1]
    loop = jnp.arange(num_nodes)
    src_f = jnp.concatenate([src, loop])
    dst_f = jnp.concatenate([dst, loop])
    w_f = jnp.concatenate([edge_weight, jnp.ones((num_nodes,), jnp.float32)])

    deg = jnp.zeros((num_nodes,), jnp.float32).at[dst_f].add(w_f)
    dinv = jnp.where(deg > 0, jax.lax.rsqrt(jnp.maximum(deg, 1e-12)), 0.0)
    norm = dinv[src_f] * w_f * dinv[dst_f]

    h = _basis_transform(x, community_assignments, basis_weights)

    hw = _matmul(h, W1)
    msg = hw[src_f] * norm[:, None]
    agg = jnp.zeros((num_nodes, H), jnp.float32).at[dst_f].add(msg)
    h = jax.nn.relu(agg + b1)
    hw = _matmul(h, W2)
    msg = hw[src_f] * norm[:, None]
    agg = jnp.zeros((num_nodes, H), jnp.float32).at[dst_f].add(msg)
    h = jax.nn.relu(agg + b2)

    scores = jax.nn.sigmoid(h @ pool_W + pool_b)[:, 0]
    k = max(1, int(0.5 * num_nodes))
    _, top_k_indices = jax.lax.top_k(scores, k)
    h_pooled = h[top_k_indices]
    batch_pooled = batch[top_k_indices]
    sums = jax.ops.segment_sum(h_pooled, batch_pooled, num_segments=NUM_GRAPHS)
    counts = jax.ops.segment_sum(jnp.ones((k,), jnp.float32), batch_pooled, num_segments=NUM_GRAPHS)
    h_global = sums / jnp.maximum(counts, 1.0)[:, None]
    h_global = jax.nn.relu(h_global @ fc1_W + fc1_b)
    out = h_global @ fc2_W + fc2_b
    return (out, top_k_indices)


# speedup vs baseline: 1.3795x; 1.3795x over previous
import functools

import jax
import jax.numpy as jnp
from jax import lax
from jax.experimental import pallas as pl
from jax.experimental.pallas import tpu as pltpu
from jax.experimental.pallas import tpu_sc as plsc

N = 10000
E = 160000
T = E + N
F_IN = 256
H = 512
C = 7
NUM_GRAPHS = 8

_BOUNDS = tuple([10656 * k for k in range(1, 7)]
                + [63936 + 10608 * m for m in range(1, 10)])

NW = 32
RPW = (N + NW - 1) // NW
CH = 16
NVR = H // 16


BM_B = 1000


def _basis_kernel(x_ref, b_ref, o_ref):
    o_ref[0] = jnp.dot(x_ref[...], b_ref[0], preferred_element_type=jnp.float32)


def _basis_all(x, basis_weights):
    return pl.pallas_call(
        _basis_kernel,
        grid=(C, N // BM_B),
        in_specs=[pl.BlockSpec((BM_B, F_IN), lambda c, m: (m, 0)),
                  pl.BlockSpec((1, F_IN, H), lambda c, m: (c, 0, 0))],
        out_specs=pl.BlockSpec((1, BM_B, H), lambda c, m: (c, m, 0)),
        out_shape=jax.ShapeDtypeStruct((C, N, H), jnp.float32),
    )(x, basis_weights)


def _mm_kernel(a_ref, b_ref, o_ref):
    o_ref[...] = jnp.dot(a_ref[...], b_ref[...], preferred_element_type=jnp.float32)


def _matmul(a, b):
    m, k = a.shape
    _, n = b.shape
    return pl.pallas_call(
        _mm_kernel,
        grid=(m // BM_B,),
        in_specs=[pl.BlockSpec((BM_B, k), lambda i: (i, 0)),
                  pl.BlockSpec((k, n), lambda i: (0, 0))],
        out_specs=pl.BlockSpec((BM_B, n), lambda i: (i, 0)),
        out_shape=jax.ShapeDtypeStruct((m, n), jnp.float32),
    )(a, b)


def _agg_body(hw_hbm, ssrc_hbm, meta_hbm, snorm_hbm, ebnd_hbm, out_hbm,
              msg_v, outbuf_v, src_v, meta_v, norm_v, bnd_v, sem):
    wid = lax.axis_index("s") * 2 + lax.axis_index("c")
    row_lo = wid * RPW
    row_hi = jnp.minimum(row_lo + RPW, N)

    pltpu.sync_copy(ebnd_hbm.at[pl.ds(wid * 8, 8)], bnd_v.at[pl.ds(0, 8)])
    bnd = bnd_v[pl.ds(0, 16)]
    e0 = bnd[0]
    e1 = bnd[1]
    base0 = (e0 // 8) * 8
    nch = (e1 - base0 + (CH - 1)) // CH

    zero = jnp.zeros((16,), jnp.float32)
    acc0 = (zero,) * NVR

    def flush(acc, cur_row, is_row_first):
        off = lax.rem(cur_row, CH) * H

        def wr(_):
            for j in range(NVR):
                outbuf_v[pl.ds(off + j * 16, 16)] = acc[j]
            return 0

        def addwr(_):
            for j in range(NVR):
                outbuf_v[pl.ds(off + j * 16, 16)] = (
                    outbuf_v[pl.ds(off + j * 16, 16)] + acc[j])
            return 0

        lax.cond(is_row_first, wr, addwr, 0)

    def chunk_body(c, carry):
        acc, cur_row, seg_open, next_dma = carry
        base = base0 + c * CH
        pltpu.sync_copy(ssrc_hbm.at[pl.ds(base, CH)], src_v)
        pltpu.sync_copy(meta_hbm.at[pl.ds(base, CH)], meta_v)
        pltpu.sync_copy(snorm_hbm.at[pl.ds(base, CH)], norm_v)
        sv = src_v[pl.ds(0, 16)]
        mv = meta_v[pl.ds(0, 16)]
        nv = norm_v[pl.ds(0, 16)]

        for g in range(CH // 8):
            handles = [
                pltpu.async_copy(
                    hw_hbm.at[pl.ds(sv[g * 8 + k] * H, H)],
                    msg_v.at[pl.ds((g * 8 + k) * H, H)], sem)
                for k in range(8)]
            for k in range(8):
                e = g * 8 + k
                handles[k].wait()
                pos = base + e
                in_rng = (pos >= e0) & (pos < e1)
                m = jnp.where(in_rng, mv[e], 0)
                nrm = jnp.where(in_rng, nv[e], 0.0)
                is_seg = (m >> 14) & 1
                is_row = (m >> 15) & 1
                row = m & 0x3FFF

                def do_flush(op, row=row, is_row=is_row):
                    acc, cur_row, seg_open = op
                    flush(acc, cur_row, seg_open == 2)
                    return (acc0, row, jnp.where(is_row == 1, 2, 1))

                def no_flush(op):
                    return op

                start = jnp.where(seg_open == 0,
                                  jnp.where(is_seg == 1, 1, 0), 0)
                acc, cur_row, seg_open = lax.cond(
                    (is_seg == 1) & (seg_open > 0), do_flush, no_flush,
                    (acc, cur_row, seg_open))
                cur_row = jnp.where(start == 1, row, cur_row)
                seg_open = jnp.where(start == 1, 2, seg_open)
                acc = tuple(
                    acc[j] + msg_v[pl.ds(e * H + j * 16, 16)] * nrm
                    for j in range(NVR))

        def dma_row(r, nd):
            pltpu.sync_copy(outbuf_v.at[pl.ds(lax.rem(r, CH) * H, H)],
                            out_hbm.at[pl.ds(r * H, H)])
            return r + 1

        next_dma = lax.fori_loop(next_dma, cur_row, dma_row, next_dma,
                                 unroll=False)
        return (acc, cur_row, seg_open, next_dma)

    acc, cur_row, seg_open, next_dma = lax.fori_loop(
        0, nch, chunk_body, (acc0, row_lo, 0, row_lo), unroll=False)

    def final_flush(op):
        acc, cur_row, seg_open = op
        flush(acc, cur_row, seg_open == 2)
        return 0

    lax.cond(seg_open > 0, final_flush, lambda op: 0, (acc, cur_row, seg_open))

    def dma_row2(r, nd):
        pltpu.sync_copy(outbuf_v.at[pl.ds(lax.rem(r, CH) * H, H)],
                        out_hbm.at[pl.ds(r * H, H)])
        return r + 1

    lax.fori_loop(next_dma, row_hi, dma_row2, next_dma, unroll=False)


def _sc_aggregate(hw, ssrc, meta, snorm, ebnd):
    mesh = plsc.VectorSubcoreMesh(core_axis_name="c", subcore_axis_name="s")
    f = pl.kernel(
        _agg_body,
        out_type=jax.ShapeDtypeStruct((N * H,), jnp.float32),
        mesh=mesh,
        scratch_types=[
            pltpu.VMEM((CH * H,), jnp.float32),
            pltpu.VMEM((CH * H,), jnp.float32),
            pltpu.VMEM((CH,), jnp.int32),
            pltpu.VMEM((CH,), jnp.int32),
            pltpu.VMEM((CH,), jnp.float32),
            pltpu.VMEM((16,), jnp.int32),
            pltpu.SemaphoreType.DMA,
        ],
    )
    return f(hw.reshape(-1), ssrc, meta, snorm, ebnd).reshape(N, H)


def kernel(x, edge_index, edge_weight, community_assignments, batch,
           basis_weights, W1, b1, W2, b2, pool_W, pool_b,
           fc1_W, fc1_b, fc2_W, fc2_b):
    num_nodes = x.shape[0]
    src, dst = edge_index[0], edge_index[1]
    loop = jnp.arange(num_nodes)
    src_f = jnp.concatenate([src, loop]).astype(jnp.int32)
    dst_f = jnp.concatenate([dst, loop]).astype(jnp.int32)
    w_f = jnp.concatenate([edge_weight, jnp.ones((num_nodes,), jnp.float32)])

    deg = jnp.zeros((num_nodes,), jnp.float32).at[dst_f].add(w_f)
    dinv = jnp.where(deg > 0, jax.lax.rsqrt(jnp.maximum(deg, 1e-12)), 0.0)
    norm = dinv[src_f] * w_f * dinv[dst_f]

    order = jnp.argsort(dst_f, stable=True)
    ssrc = src_f[order]
    snorm = norm[order]
    srow = dst_f[order]
    new_row = jnp.concatenate([jnp.ones((1,), jnp.int32),
                               (srow[1:] != srow[:-1]).astype(jnp.int32)])
    forced = jnp.zeros((T,), jnp.int32).at[jnp.array(_BOUNDS)].set(1)
    new_seg = new_row | forced
    meta = srow | (new_seg << 14) | (new_row << 15)
    pad = CH
    ssrc = jnp.concatenate([ssrc, jnp.zeros((pad,), jnp.int32)])
    snorm = jnp.concatenate([snorm, jnp.zeros((pad,), jnp.float32)])
    meta = jnp.concatenate([meta, jnp.zeros((pad,), jnp.int32)])
    row_ptr = jnp.concatenate([
        jnp.zeros((1,), jnp.int32),
        jnp.cumsum(jnp.zeros((num_nodes,), jnp.int32).at[dst_f].add(1))
        .astype(jnp.int32)])
    wrk = jnp.arange(NW, dtype=jnp.int32)
    estart = row_ptr[wrk * RPW]
    eend = row_ptr[jnp.minimum(wrk * RPW + RPW, num_nodes)]
    ebnd = jnp.zeros((NW, 8), jnp.int32).at[:, 0].set(estart).at[:, 1].set(eend).reshape(-1)

    all_h = _basis_all(x, basis_weights)
    h = all_h[community_assignments, jnp.arange(num_nodes)]

    hw = _matmul(h, W1)
    agg = _sc_aggregate(hw, ssrc, meta, snorm, ebnd)
    h = jax.nn.relu(agg + b1)
    hw = _matmul(h, W2)
    agg = _sc_aggregate(hw, ssrc, meta, snorm, ebnd)
    h = jax.nn.relu(agg + b2)

    scores = jax.nn.sigmoid(h @ pool_W + pool_b)[:, 0]
    k = max(1, int(0.5 * num_nodes))
    _, top_k_indices = jax.lax.top_k(scores, k)
    h_pooled = h[top_k_indices]
    batch_pooled = batch[top_k_indices]
    sums = jax.ops.segment_sum(h_pooled, batch_pooled, num_segments=NUM_GRAPHS)
    counts = jax.ops.segment_sum(jnp.ones((k,), jnp.float32), batch_pooled,
                                 num_segments=NUM_GRAPHS)
    h_global = sums / jnp.maximum(counts, 1.0)[:, None]
    h_global = jax.nn.relu(h_global @ fc1_W + fc1_b)
    out = h_global @ fc2_W + fc2_b
    return (out, top_k_indices)


# speedup vs baseline: 1.5830x; 1.1475x over previous
import functools

import jax
import jax.numpy as jnp
from jax import lax
from jax.experimental import pallas as pl
from jax.experimental.pallas import tpu as pltpu
from jax.experimental.pallas import tpu_sc as plsc

N = 10000
E = 160000
T = E + N
F_IN = 256
H = 512
C = 7
NUM_GRAPHS = 8

_BOUNDS = tuple([10656 * k for k in range(1, 7)]
                + [63936 + 10608 * m for m in range(1, 10)])

NW = 32
RPW = (N + NW - 1) // NW
CH = 16
NVR = H // 16


BM_B = 1000


def _basis_kernel(x_ref, b_ref, o_ref):
    o_ref[0] = jnp.dot(x_ref[...], b_ref[0], preferred_element_type=jnp.float32)


def _basis_all(x, basis_weights):
    return pl.pallas_call(
        _basis_kernel,
        grid=(C, N // BM_B),
        in_specs=[pl.BlockSpec((BM_B, F_IN), lambda c, m: (m, 0)),
                  pl.BlockSpec((1, F_IN, H), lambda c, m: (c, 0, 0))],
        out_specs=pl.BlockSpec((1, BM_B, H), lambda c, m: (c, m, 0)),
        out_shape=jax.ShapeDtypeStruct((C, N, H), jnp.float32),
    )(x, basis_weights)


def _mm_kernel(a_ref, b_ref, o_ref):
    o_ref[...] = jnp.dot(a_ref[...], b_ref[...], preferred_element_type=jnp.float32)


def _matmul(a, b):
    m, k = a.shape
    _, n = b.shape
    return pl.pallas_call(
        _mm_kernel,
        grid=(m // BM_B,),
        in_specs=[pl.BlockSpec((BM_B, k), lambda i: (i, 0)),
                  pl.BlockSpec((k, n), lambda i: (0, 0))],
        out_specs=pl.BlockSpec((BM_B, n), lambda i: (i, 0)),
        out_shape=jax.ShapeDtypeStruct((m, n), jnp.float32),
    )(a, b)


def _agg_body(hw_hbm, ssrc_hbm, meta_hbm, snorm_hbm, ebnd_hbm, out_hbm,
              msg_v, outbuf_v, src_v, meta_v, norm_v, bnd_v, sem):
    wid = lax.axis_index("s") * 2 + lax.axis_index("c")
    row_lo = wid * RPW
    row_hi = jnp.minimum(row_lo + RPW, N)

    pltpu.sync_copy(ebnd_hbm.at[pl.ds(wid * 8, 8)], bnd_v.at[pl.ds(0, 8)])
    bnd = bnd_v[pl.ds(0, 16)]
    e0 = bnd[0]
    e1 = bnd[1]
    base0 = (e0 // 8) * 8
    nch = (e1 - base0 + (CH - 1)) // CH

    zero = jnp.zeros((16,), jnp.float32)
    acc0 = (zero,) * NVR

    def flush(acc, cur_row, is_row_first):
        off = lax.rem(cur_row, CH) * H

        def wr(_):
            for j in range(NVR):
                outbuf_v[pl.ds(off + j * 16, 16)] = acc[j]
            return 0

        def addwr(_):
            for j in range(NVR):
                outbuf_v[pl.ds(off + j * 16, 16)] = (
                    outbuf_v[pl.ds(off + j * 16, 16)] + acc[j])
            return 0

        lax.cond(is_row_first, wr, addwr, 0)

    def chunk_body(c, carry):
        acc, cur_row, seg_open, next_dma = carry
        base = base0 + c * CH
        hm = [pltpu.async_copy(ssrc_hbm.at[pl.ds(base, CH)], src_v, sem),
              pltpu.async_copy(meta_hbm.at[pl.ds(base, CH)], meta_v, sem),
              pltpu.async_copy(snorm_hbm.at[pl.ds(base, CH)], norm_v, sem)]
        for h in hm:
            h.wait()
        sv = src_v[pl.ds(0, 16)]
        mv = meta_v[pl.ds(0, 16)]
        nv = norm_v[pl.ds(0, 16)]

        handles = [
            pltpu.async_copy(
                hw_hbm.at[pl.ds(sv[e] * H, H)],
                msg_v.at[pl.ds(e * H, H)], sem)
            for e in range(CH)]
        for h in handles:
            h.wait()
        for g in range(CH // 8):
            for k in range(8):
                e = g * 8 + k
                pos = base + e
                in_rng = (pos >= e0) & (pos < e1)
                m = jnp.where(in_rng, mv[e], 0)
                nrm = jnp.where(in_rng, nv[e], 0.0)
                is_seg = (m >> 14) & 1
                is_row = (m >> 15) & 1
                row = m & 0x3FFF

                def do_flush(op, row=row, is_row=is_row):
                    acc, cur_row, seg_open = op
                    flush(acc, cur_row, seg_open == 2)
                    return (acc0, row, jnp.where(is_row == 1, 2, 1))

                def no_flush(op):
                    return op

                start = jnp.where(seg_open == 0,
                                  jnp.where(is_seg == 1, 1, 0), 0)
                acc, cur_row, seg_open = lax.cond(
                    (is_seg == 1) & (seg_open > 0), do_flush, no_flush,
                    (acc, cur_row, seg_open))
                cur_row = jnp.where(start == 1, row, cur_row)
                seg_open = jnp.where(start == 1, 2, seg_open)
                acc = tuple(
                    acc[j] + msg_v[pl.ds(e * H + j * 16, 16)] * nrm
                    for j in range(NVR))

        def dma_row(r, nd):
            pltpu.sync_copy(outbuf_v.at[pl.ds(lax.rem(r, CH) * H, H)],
                            out_hbm.at[pl.ds(r * H, H)])
            return r + 1

        next_dma = lax.fori_loop(next_dma, cur_row, dma_row, next_dma,
                                 unroll=False)
        return (acc, cur_row, seg_open, next_dma)

    acc, cur_row, seg_open, next_dma = lax.fori_loop(
        0, nch, chunk_body, (acc0, row_lo, 0, row_lo), unroll=False)

    def final_flush(op):
        acc, cur_row, seg_open = op
        flush(acc, cur_row, seg_open == 2)
        return 0

    lax.cond(seg_open > 0, final_flush, lambda op: 0, (acc, cur_row, seg_open))

    def dma_row2(r, nd):
        pltpu.sync_copy(outbuf_v.at[pl.ds(lax.rem(r, CH) * H, H)],
                        out_hbm.at[pl.ds(r * H, H)])
        return r + 1

    lax.fori_loop(next_dma, row_hi, dma_row2, next_dma, unroll=False)


def _sc_aggregate(hw, ssrc, meta, snorm, ebnd):
    mesh = plsc.VectorSubcoreMesh(core_axis_name="c", subcore_axis_name="s")
    f = pl.kernel(
        _agg_body,
        out_type=jax.ShapeDtypeStruct((N * H,), jnp.float32),
        mesh=mesh,
        scratch_types=[
            pltpu.VMEM((CH * H,), jnp.float32),
            pltpu.VMEM((CH * H,), jnp.float32),
            pltpu.VMEM((CH,), jnp.int32),
            pltpu.VMEM((CH,), jnp.int32),
            pltpu.VMEM((CH,), jnp.float32),
            pltpu.VMEM((16,), jnp.int32),
            pltpu.SemaphoreType.DMA,
        ],
    )
    return f(hw.reshape(-1), ssrc, meta, snorm, ebnd).reshape(N, H)


def kernel(x, edge_index, edge_weight, community_assignments, batch,
           basis_weights, W1, b1, W2, b2, pool_W, pool_b,
           fc1_W, fc1_b, fc2_W, fc2_b):
    num_nodes = x.shape[0]
    src, dst = edge_index[0], edge_index[1]
    loop = jnp.arange(num_nodes)
    src_f = jnp.concatenate([src, loop]).astype(jnp.int32)
    dst_f = jnp.concatenate([dst, loop]).astype(jnp.int32)
    w_f = jnp.concatenate([edge_weight, jnp.ones((num_nodes,), jnp.float32)])

    deg = jnp.zeros((num_nodes,), jnp.float32).at[dst_f].add(w_f)
    dinv = jnp.where(deg > 0, jax.lax.rsqrt(jnp.maximum(deg, 1e-12)), 0.0)
    norm = dinv[src_f] * w_f * dinv[dst_f]

    order = jnp.argsort(dst_f, stable=True)
    ssrc = src_f[order]
    snorm = norm[order]
    srow = dst_f[order]
    new_row = jnp.concatenate([jnp.ones((1,), jnp.int32),
                               (srow[1:] != srow[:-1]).astype(jnp.int32)])
    forced = jnp.zeros((T,), jnp.int32).at[jnp.array(_BOUNDS)].set(1)
    new_seg = new_row | forced
    meta = srow | (new_seg << 14) | (new_row << 15)
    pad = CH
    ssrc = jnp.concatenate([ssrc, jnp.zeros((pad,), jnp.int32)])
    snorm = jnp.concatenate([snorm, jnp.zeros((pad,), jnp.float32)])
    meta = jnp.concatenate([meta, jnp.zeros((pad,), jnp.int32)])
    row_ptr = jnp.concatenate([
        jnp.zeros((1,), jnp.int32),
        jnp.cumsum(jnp.zeros((num_nodes,), jnp.int32).at[dst_f].add(1))
        .astype(jnp.int32)])
    wrk = jnp.arange(NW, dtype=jnp.int32)
    estart = row_ptr[wrk * RPW]
    eend = row_ptr[jnp.minimum(wrk * RPW + RPW, num_nodes)]
    ebnd = jnp.zeros((NW, 8), jnp.int32).at[:, 0].set(estart).at[:, 1].set(eend).reshape(-1)

    all_h = _basis_all(x, basis_weights)
    h = all_h[community_assignments, jnp.arange(num_nodes)]

    hw = _matmul(h, W1)
    agg = _sc_aggregate(hw, ssrc, meta, snorm, ebnd)
    h = jax.nn.relu(agg + b1)
    hw = _matmul(h, W2)
    agg = _sc_aggregate(hw, ssrc, meta, snorm, ebnd)
    h = jax.nn.relu(agg + b2)

    scores = jax.nn.sigmoid(h @ pool_W + pool_b)[:, 0]
    k = max(1, int(0.5 * num_nodes))
    _, top_k_indices = jax.lax.top_k(scores, k)
    h_pooled = h[top_k_indices]
    batch_pooled = batch[top_k_indices]
    sums = jax.ops.segment_sum(h_pooled, batch_pooled, num_segments=NUM_GRAPHS)
    counts = jax.ops.segment_sum(jnp.ones((k,), jnp.float32), batch_pooled,
                                 num_segments=NUM_GRAPHS)
    h_global = sums / jnp.maximum(counts, 1.0)[:, None]
    h_global = jax.nn.relu(h_global @ fc1_W + fc1_b)
    out = h_global @ fc2_W + fc2_b
    return (out, top_k_indices)


# speedup vs baseline: 1.6207x; 1.0238x over previous
import functools

import jax
import jax.numpy as jnp
from jax import lax
from jax.experimental import pallas as pl
from jax.experimental.pallas import tpu as pltpu
from jax.experimental.pallas import tpu_sc as plsc

N = 10000
E = 160000
T = E + N
F_IN = 256
H = 512
C = 7
NUM_GRAPHS = 8

_BOUNDS = tuple([10656 * k for k in range(1, 7)]
                + [63936 + 10608 * m for m in range(1, 10)])

NW = 32
RPW = (N + NW - 1) // NW
CH = 32
NVR = H // 16


BM_B = 1000


def _basis_kernel(x_ref, b_ref, o_ref):
    o_ref[0] = jnp.dot(x_ref[...], b_ref[0], preferred_element_type=jnp.float32)


def _basis_all(x, basis_weights):
    return pl.pallas_call(
        _basis_kernel,
        grid=(C, N // BM_B),
        in_specs=[pl.BlockSpec((BM_B, F_IN), lambda c, m: (m, 0)),
                  pl.BlockSpec((1, F_IN, H), lambda c, m: (c, 0, 0))],
        out_specs=pl.BlockSpec((1, BM_B, H), lambda c, m: (c, m, 0)),
        out_shape=jax.ShapeDtypeStruct((C, N, H), jnp.float32),
    )(x, basis_weights)


def _mm_kernel(a_ref, b_ref, o_ref):
    o_ref[...] = jnp.dot(a_ref[...], b_ref[...], preferred_element_type=jnp.float32)


def _matmul(a, b):
    m, k = a.shape
    _, n = b.shape
    return pl.pallas_call(
        _mm_kernel,
        grid=(m // BM_B,),
        in_specs=[pl.BlockSpec((BM_B, k), lambda i: (i, 0)),
                  pl.BlockSpec((k, n), lambda i: (0, 0))],
        out_specs=pl.BlockSpec((BM_B, n), lambda i: (i, 0)),
        out_shape=jax.ShapeDtypeStruct((m, n), jnp.float32),
    )(a, b)


def _agg_body(hw_hbm, ssrc_hbm, meta_hbm, snorm_hbm, ebnd_hbm, out_hbm,
              msg_v, outbuf_v, src_v, meta_v, norm_v, bnd_v, sem):
    wid = lax.axis_index("s") * 2 + lax.axis_index("c")
    row_lo = wid * RPW
    row_hi = jnp.minimum(row_lo + RPW, N)

    pltpu.sync_copy(ebnd_hbm.at[pl.ds(wid * 8, 8)], bnd_v.at[pl.ds(0, 8)])
    bnd = bnd_v[pl.ds(0, 16)]
    e0 = bnd[0]
    e1 = bnd[1]
    base0 = (e0 // 8) * 8
    nch = (e1 - base0 + (CH - 1)) // CH

    zero = jnp.zeros((16,), jnp.float32)
    acc0 = (zero,) * NVR

    def flush(acc, cur_row, is_row_first):
        off = lax.rem(cur_row, CH) * H

        def wr(_):
            for j in range(NVR):
                outbuf_v[pl.ds(off + j * 16, 16)] = acc[j]
            return 0

        def addwr(_):
            for j in range(NVR):
                outbuf_v[pl.ds(off + j * 16, 16)] = (
                    outbuf_v[pl.ds(off + j * 16, 16)] + acc[j])
            return 0

        lax.cond(is_row_first, wr, addwr, 0)

    def chunk_body(c, carry):
        acc, cur_row, seg_open, next_dma = carry
        base = base0 + c * CH
        hm = [pltpu.async_copy(ssrc_hbm.at[pl.ds(base, CH)], src_v, sem),
              pltpu.async_copy(meta_hbm.at[pl.ds(base, CH)], meta_v, sem),
              pltpu.async_copy(snorm_hbm.at[pl.ds(base, CH)], norm_v, sem)]
        for h in hm:
            h.wait()
        svs = [src_v[pl.ds(0, 16)], src_v[pl.ds(16, 16)]]
        mvs = [meta_v[pl.ds(0, 16)], meta_v[pl.ds(16, 16)]]
        nvs = [norm_v[pl.ds(0, 16)], norm_v[pl.ds(16, 16)]]

        handles = [
            pltpu.async_copy(
                hw_hbm.at[pl.ds(svs[e // 16][e % 16] * H, H)],
                msg_v.at[pl.ds(e * H, H)], sem)
            for e in range(CH)]
        for h in handles:
            h.wait()
        for g in range(CH // 8):
            for k in range(8):
                e = g * 8 + k
                pos = base + e
                in_rng = (pos >= e0) & (pos < e1)
                m = jnp.where(in_rng, mvs[e // 16][e % 16], 0)
                nrm = jnp.where(in_rng, nvs[e // 16][e % 16], 0.0)
                is_seg = (m >> 14) & 1
                is_row = (m >> 15) & 1
                row = m & 0x3FFF

                def do_flush(op, row=row, is_row=is_row):
                    acc, cur_row, seg_open = op
                    flush(acc, cur_row, seg_open == 2)
                    return (acc0, row, jnp.where(is_row == 1, 2, 1))

                def no_flush(op):
                    return op

                start = jnp.where(seg_open == 0,
                                  jnp.where(is_seg == 1, 1, 0), 0)
                acc, cur_row, seg_open = lax.cond(
                    (is_seg == 1) & (seg_open > 0), do_flush, no_flush,
                    (acc, cur_row, seg_open))
                cur_row = jnp.where(start == 1, row, cur_row)
                seg_open = jnp.where(start == 1, 2, seg_open)
                acc = tuple(
                    acc[j] + msg_v[pl.ds(e * H + j * 16, 16)] * nrm
                    for j in range(NVR))

        def dma_row(r, nd):
            pltpu.sync_copy(outbuf_v.at[pl.ds(lax.rem(r, CH) * H, H)],
                            out_hbm.at[pl.ds(r * H, H)])
            return r + 1

        next_dma = lax.fori_loop(next_dma, cur_row, dma_row, next_dma,
                                 unroll=False)
        return (acc, cur_row, seg_open, next_dma)

    acc, cur_row, seg_open, next_dma = lax.fori_loop(
        0, nch, chunk_body, (acc0, row_lo, 0, row_lo), unroll=False)

    def final_flush(op):
        acc, cur_row, seg_open = op
        flush(acc, cur_row, seg_open == 2)
        return 0

    lax.cond(seg_open > 0, final_flush, lambda op: 0, (acc, cur_row, seg_open))

    def dma_row2(r, nd):
        pltpu.sync_copy(outbuf_v.at[pl.ds(lax.rem(r, CH) * H, H)],
                        out_hbm.at[pl.ds(r * H, H)])
        return r + 1

    lax.fori_loop(next_dma, row_hi, dma_row2, next_dma, unroll=False)


def _sc_aggregate(hw, ssrc, meta, snorm, ebnd):
    mesh = plsc.VectorSubcoreMesh(core_axis_name="c", subcore_axis_name="s")
    f = pl.kernel(
        _agg_body,
        out_type=jax.ShapeDtypeStruct((N * H,), jnp.float32),
        mesh=mesh,
        scratch_types=[
            pltpu.VMEM((CH * H,), jnp.float32),
            pltpu.VMEM((CH * H,), jnp.float32),
            pltpu.VMEM((CH,), jnp.int32),
            pltpu.VMEM((CH,), jnp.int32),
            pltpu.VMEM((CH,), jnp.float32),
            pltpu.VMEM((16,), jnp.int32),
            pltpu.SemaphoreType.DMA,
        ],
    )
    return f(hw.reshape(-1), ssrc, meta, snorm, ebnd).reshape(N, H)


def kernel(x, edge_index, edge_weight, community_assignments, batch,
           basis_weights, W1, b1, W2, b2, pool_W, pool_b,
           fc1_W, fc1_b, fc2_W, fc2_b):
    num_nodes = x.shape[0]
    src, dst = edge_index[0], edge_index[1]
    loop = jnp.arange(num_nodes)
    src_f = jnp.concatenate([src, loop]).astype(jnp.int32)
    dst_f = jnp.concatenate([dst, loop]).astype(jnp.int32)
    w_f = jnp.concatenate([edge_weight, jnp.ones((num_nodes,), jnp.float32)])

    deg = jnp.zeros((num_nodes,), jnp.float32).at[dst_f].add(w_f)
    dinv = jnp.where(deg > 0, jax.lax.rsqrt(jnp.maximum(deg, 1e-12)), 0.0)
    norm = dinv[src_f] * w_f * dinv[dst_f]

    order = jnp.argsort(dst_f, stable=True)
    ssrc = src_f[order]
    snorm = norm[order]
    srow = dst_f[order]
    new_row = jnp.concatenate([jnp.ones((1,), jnp.int32),
                               (srow[1:] != srow[:-1]).astype(jnp.int32)])
    forced = jnp.zeros((T,), jnp.int32).at[jnp.array(_BOUNDS)].set(1)
    new_seg = new_row | forced
    meta = srow | (new_seg << 14) | (new_row << 15)
    pad = CH
    ssrc = jnp.concatenate([ssrc, jnp.zeros((pad,), jnp.int32)])
    snorm = jnp.concatenate([snorm, jnp.zeros((pad,), jnp.float32)])
    meta = jnp.concatenate([meta, jnp.zeros((pad,), jnp.int32)])
    row_ptr = jnp.concatenate([
        jnp.zeros((1,), jnp.int32),
        jnp.cumsum(jnp.zeros((num_nodes,), jnp.int32).at[dst_f].add(1))
        .astype(jnp.int32)])
    wrk = jnp.arange(NW, dtype=jnp.int32)
    estart = row_ptr[wrk * RPW]
    eend = row_ptr[jnp.minimum(wrk * RPW + RPW, num_nodes)]
    ebnd = jnp.zeros((NW, 8), jnp.int32).at[:, 0].set(estart).at[:, 1].set(eend).reshape(-1)

    all_h = _basis_all(x, basis_weights)
    h = all_h[community_assignments, jnp.arange(num_nodes)]

    hw = _matmul(h, W1)
    agg = _sc_aggregate(hw, ssrc, meta, snorm, ebnd)
    h = jax.nn.relu(agg + b1)
    hw = _matmul(h, W2)
    agg = _sc_aggregate(hw, ssrc, meta, snorm, ebnd)
    h = jax.nn.relu(agg + b2)

    scores = jax.nn.sigmoid(h @ pool_W + pool_b)[:, 0]
    k = max(1, int(0.5 * num_nodes))
    _, top_k_indices = jax.lax.top_k(scores, k)
    h_pooled = h[top_k_indices]
    batch_pooled = batch[top_k_indices]
    sums = jax.ops.segment_sum(h_pooled, batch_pooled, num_segments=NUM_GRAPHS)
    counts = jax.ops.segment_sum(jnp.ones((k,), jnp.float32), batch_pooled,
                                 num_segments=NUM_GRAPHS)
    h_global = sums / jnp.maximum(counts, 1.0)[:, None]
    h_global = jax.nn.relu(h_global @ fc1_W + fc1_b)
    out = h_global @ fc2_W + fc2_b
    return (out, top_k_indices)


# speedup vs baseline: 1.6211x; 1.0003x over previous
import jax
import jax.numpy as jnp
from jax import lax
from jax.experimental import pallas as pl
from jax.experimental.pallas import tpu as pltpu
from jax.experimental.pallas import tpu_sc as plsc

N = 10000
E = 160000
T = E + N
F_IN = 256
H = 512
C = 7
NUM_GRAPHS = 8

_BOUNDS = tuple([10656 * k for k in range(1, 7)]
                + [63936 + 10608 * m for m in range(1, 10)])

NW = 32
RPW = (N + NW - 1) // NW
CH = 32
NVR = H // 16


BM_B = 1000


def _basis_kernel(x_ref, b_ref, o_ref):
    o_ref[0] = jnp.dot(x_ref[...], b_ref[0], preferred_element_type=jnp.float32)


def _basis_all(x, basis_weights):
    return pl.pallas_call(
        _basis_kernel,
        grid=(C, N // BM_B),
        in_specs=[pl.BlockSpec((BM_B, F_IN), lambda c, m: (m, 0)),
                  pl.BlockSpec((1, F_IN, H), lambda c, m: (c, 0, 0))],
        out_specs=pl.BlockSpec((1, BM_B, H), lambda c, m: (c, m, 0)),
        out_shape=jax.ShapeDtypeStruct((C, N, H), jnp.float32),
    )(x, basis_weights)


def _mm_kernel(a_ref, b_ref, o_ref):
    o_ref[...] = jnp.dot(a_ref[...], b_ref[...], preferred_element_type=jnp.float32)


def _matmul(a, b):
    m, k = a.shape
    _, n = b.shape
    return pl.pallas_call(
        _mm_kernel,
        grid=(m // BM_B,),
        in_specs=[pl.BlockSpec((BM_B, k), lambda i: (i, 0)),
                  pl.BlockSpec((k, n), lambda i: (0, 0))],
        out_specs=pl.BlockSpec((BM_B, n), lambda i: (i, 0)),
        out_shape=jax.ShapeDtypeStruct((m, n), jnp.float32),
    )(a, b)


def _agg_body(hw_hbm, ssrc_hbm, meta_hbm, snorm_hbm, ebnd_hbm, out_hbm,
              msg_v, outbuf_v, src_v, meta_v, norm_v, bnd_v, sem):
    wid = lax.axis_index("s") * 2 + lax.axis_index("c")
    row_lo = wid * RPW
    row_hi = jnp.minimum(row_lo + RPW, N)

    pltpu.sync_copy(ebnd_hbm.at[pl.ds(wid * 8, 8)], bnd_v.at[pl.ds(0, 8)])
    bnd = bnd_v[pl.ds(0, 16)]
    e0 = bnd[0]
    e1 = bnd[1]
    base0 = (e0 // 8) * 8
    nch = (e1 - base0 + (CH - 1)) // CH

    zero = jnp.zeros((16,), jnp.float32)
    acc0 = (zero,) * NVR

    def flush(acc, cur_row, is_row_first):
        off = lax.rem(cur_row, CH) * H

        def wr(_):
            for j in range(NVR):
                outbuf_v[pl.ds(off + j * 16, 16)] = acc[j]
            return 0

        def addwr(_):
            for j in range(NVR):
                outbuf_v[pl.ds(off + j * 16, 16)] = (
                    outbuf_v[pl.ds(off + j * 16, 16)] + acc[j])
            return 0

        lax.cond(is_row_first, wr, addwr, 0)

    def chunk_body(c, carry):
        acc, cur_row, seg_open, next_dma = carry
        base = base0 + c * CH
        hm = [pltpu.async_copy(ssrc_hbm.at[pl.ds(base, CH)], src_v, sem),
              pltpu.async_copy(meta_hbm.at[pl.ds(base, CH)], meta_v, sem),
              pltpu.async_copy(snorm_hbm.at[pl.ds(base, CH)], norm_v, sem)]
        for h in hm:
            h.wait()
        svs = [src_v[pl.ds(0, 16)], src_v[pl.ds(16, 16)]]
        mvs = [meta_v[pl.ds(0, 16)], meta_v[pl.ds(16, 16)]]
        nvs = [norm_v[pl.ds(0, 16)], norm_v[pl.ds(16, 16)]]

        handles = [
            pltpu.async_copy(
                hw_hbm.at[pl.ds(svs[e // 16][e % 16] * H, H)],
                msg_v.at[pl.ds(e * H, H)], sem)
            for e in range(CH)]
        for h in handles:
            h.wait()
        for g in range(CH // 8):
            for k in range(8):
                e = g * 8 + k
                pos = base + e
                in_rng = (pos >= e0) & (pos < e1)
                m = jnp.where(in_rng, mvs[e // 16][e % 16], 0)
                nrm = jnp.where(in_rng, nvs[e // 16][e % 16], 0.0)
                is_seg = (m >> 14) & 1
                is_row = (m >> 15) & 1
                row = m & 0x3FFF

                def do_flush(op, row=row, is_row=is_row):
                    acc, cur_row, seg_open = op
                    flush(acc, cur_row, seg_open == 2)
                    return (acc0, row, jnp.where(is_row == 1, 2, 1))

                def no_flush(op):
                    return op

                start = jnp.where(seg_open == 0,
                                  jnp.where(is_seg == 1, 1, 0), 0)
                acc, cur_row, seg_open = lax.cond(
                    (is_seg == 1) & (seg_open > 0), do_flush, no_flush,
                    (acc, cur_row, seg_open))
                cur_row = jnp.where(start == 1, row, cur_row)
                seg_open = jnp.where(start == 1, 2, seg_open)
                acc = tuple(
                    acc[j] + msg_v[pl.ds(e * H + j * 16, 16)] * nrm
                    for j in range(NVR))

        def dma_row(r, nd):
            pltpu.sync_copy(outbuf_v.at[pl.ds(lax.rem(r, CH) * H, H)],
                            out_hbm.at[pl.ds(r * H, H)])
            return r + 1

        next_dma = lax.fori_loop(next_dma, cur_row, dma_row, next_dma,
                                 unroll=False)
        return (acc, cur_row, seg_open, next_dma)

    acc, cur_row, seg_open, next_dma = lax.fori_loop(
        0, nch, chunk_body, (acc0, row_lo, 0, row_lo), unroll=False)

    def final_flush(op):
        acc, cur_row, seg_open = op
        flush(acc, cur_row, seg_open == 2)
        return 0

    lax.cond(seg_open > 0, final_flush, lambda op: 0, (acc, cur_row, seg_open))

    def dma_row2(r, nd):
        pltpu.sync_copy(outbuf_v.at[pl.ds(lax.rem(r, CH) * H, H)],
                        out_hbm.at[pl.ds(r * H, H)])
        return r + 1

    lax.fori_loop(next_dma, row_hi, dma_row2, next_dma, unroll=False)


def _sc_aggregate(hw, ssrc, meta, snorm, ebnd):
    mesh = plsc.VectorSubcoreMesh(core_axis_name="c", subcore_axis_name="s")
    f = pl.kernel(
        _agg_body,
        out_type=jax.ShapeDtypeStruct((N * H,), jnp.float32),
        mesh=mesh,
        scratch_types=[
            pltpu.VMEM((CH * H,), jnp.float32),
            pltpu.VMEM((CH * H,), jnp.float32),
            pltpu.VMEM((CH,), jnp.int32),
            pltpu.VMEM((CH,), jnp.int32),
            pltpu.VMEM((CH,), jnp.float32),
            pltpu.VMEM((16,), jnp.int32),
            pltpu.SemaphoreType.DMA,
        ],
    )
    return f(hw.reshape(-1), ssrc, meta, snorm, ebnd).reshape(N, H)


def kernel(x, edge_index, edge_weight, community_assignments, batch,
           basis_weights, W1, b1, W2, b2, pool_W, pool_b,
           fc1_W, fc1_b, fc2_W, fc2_b):
    num_nodes = x.shape[0]
    src, dst = edge_index[0], edge_index[1]
    loop = jnp.arange(num_nodes)
    src_f = jnp.concatenate([src, loop]).astype(jnp.int32)
    dst_f = jnp.concatenate([dst, loop]).astype(jnp.int32)
    w_f = jnp.concatenate([edge_weight, jnp.ones((num_nodes,), jnp.float32)])

    deg = jnp.zeros((num_nodes,), jnp.float32).at[dst_f].add(w_f)
    dinv = jnp.where(deg > 0, jax.lax.rsqrt(jnp.maximum(deg, 1e-12)), 0.0)
    norm = dinv[src_f] * w_f * dinv[dst_f]

    order = jnp.argsort(dst_f, stable=True)
    ssrc = src_f[order]
    snorm = norm[order]
    srow = dst_f[order]
    new_row = jnp.concatenate([jnp.ones((1,), jnp.int32),
                               (srow[1:] != srow[:-1]).astype(jnp.int32)])
    forced = jnp.zeros((T,), jnp.int32).at[jnp.array(_BOUNDS)].set(1)
    new_seg = new_row | forced
    meta = srow | (new_seg << 14) | (new_row << 15)
    pad = CH
    ssrc = jnp.concatenate([ssrc, jnp.zeros((pad,), jnp.int32)])
    snorm = jnp.concatenate([snorm, jnp.zeros((pad,), jnp.float32)])
    meta = jnp.concatenate([meta, jnp.zeros((pad,), jnp.int32)])
    row_ptr = jnp.concatenate([
        jnp.zeros((1,), jnp.int32),
        jnp.cumsum(jnp.zeros((num_nodes,), jnp.int32).at[dst_f].add(1))
        .astype(jnp.int32)])
    wrk = jnp.arange(NW, dtype=jnp.int32)
    estart = row_ptr[wrk * RPW]
    eend = row_ptr[jnp.minimum(wrk * RPW + RPW, num_nodes)]
    ebnd = jnp.zeros((NW, 8), jnp.int32).at[:, 0].set(estart).at[:, 1].set(eend).reshape(-1)

    all_h = _basis_all(x, basis_weights)
    h = all_h[community_assignments, jnp.arange(num_nodes)]

    hw = _matmul(h, W1)
    agg = _sc_aggregate(hw, ssrc, meta, snorm, ebnd)
    h = jax.nn.relu(agg + b1)
    hw = _matmul(h, W2)
    agg = _sc_aggregate(hw, ssrc, meta, snorm, ebnd)
    h = jax.nn.relu(agg + b2)

    scores = jax.nn.sigmoid(h @ pool_W + pool_b)[:, 0]
    k = max(1, int(0.5 * num_nodes))
    _, top_k_indices = jax.lax.top_k(scores, k)
    h_pooled = h[top_k_indices]
    batch_pooled = batch[top_k_indices]
    sums = jax.ops.segment_sum(h_pooled, batch_pooled, num_segments=NUM_GRAPHS)
    counts = jax.ops.segment_sum(jnp.ones((k,), jnp.float32), batch_pooled,
                                 num_segments=NUM_GRAPHS)
    h_global = sums / jnp.maximum(counts, 1.0)[:, None]
    h_global = jax.nn.relu(h_global @ fc1_W + fc1_b)
    out = h_global @ fc2_W + fc2_b
    return (out, top_k_indices)
